# Initial kernel scaffold; baseline (speedup 1.0000x reference)
import jax
import jax.numpy as jnp
from jax.experimental import pallas as pl


def kernel(node_edge_feat, dist_feat, srcs, dsts, W, b):
    raise NotImplementedError("write your pallas kernel here")


# speedup vs baseline: 3.3100x; 3.3100x over previous
import functools

import jax
import jax.numpy as jnp
from jax import lax
from jax.experimental import pallas as pl
from jax.experimental.pallas import tpu as pltpu
from jax.experimental.pallas import tpu_sc as plsc

NUM_NODES = 10000
NUM_EDGES = 320000
HIDDEN = 128

_NC = 2
_NS = 16
_NW = _NC * _NS

_CHUNK = 128
_PER_W_CHUNKS = 79
_PER_W = _PER_W_CHUNKS * _CHUNK
_B_PAD = _PER_W * _NW


def _gather_body(table_hbm, srcs_hbm, dsts_hbm, out_src_hbm, out_dst_hbm,
                 idx_s, idx_d, rows_s, rows_d, sem_s, sem_d):
    wid = lax.axis_index("s") * _NC + lax.axis_index("c")
    base = wid * _PER_W

    def step(g, carry):
        off = base + g * _CHUNK
        pltpu.sync_copy(srcs_hbm.at[pl.ds(off, _CHUNK)], idx_s)
        pltpu.sync_copy(dsts_hbm.at[pl.ds(off, _CHUNK)], idx_d)
        cp_s = pltpu.async_copy(table_hbm.at[idx_s], rows_s, sem_s)
        cp_d = pltpu.async_copy(table_hbm.at[idx_d], rows_d, sem_d)
        cp_s.wait()
        pltpu.sync_copy(rows_s, out_src_hbm.at[pl.ds(off, _CHUNK)])
        cp_d.wait()
        pltpu.sync_copy(rows_d, out_dst_hbm.at[pl.ds(off, _CHUNK)])
        return carry

    lax.fori_loop(0, _PER_W_CHUNKS, step, 0)


_sc_gather = functools.partial(
    pl.kernel,
    mesh=plsc.VectorSubcoreMesh(core_axis_name="c", subcore_axis_name="s"),
    out_type=[
        jax.ShapeDtypeStruct((_B_PAD, HIDDEN), jnp.float32),
        jax.ShapeDtypeStruct((_B_PAD, HIDDEN), jnp.float32),
    ],
    scratch_types=[
        pltpu.VMEM((_CHUNK,), jnp.int32),
        pltpu.VMEM((_CHUNK,), jnp.int32),
        pltpu.VMEM((_CHUNK, HIDDEN), jnp.float32),
        pltpu.VMEM((_CHUNK, HIDDEN), jnp.float32),
        pltpu.SemaphoreType.DMA,
        pltpu.SemaphoreType.DMA,
    ],
)(_gather_body)


_BLK = 3200


def _mm_body(src_ref, dst_ref, dist_ref, ws_ref, wd_ref, we_ref, b_ref, o_ref):
    acc = jnp.dot(src_ref[...], ws_ref[...], preferred_element_type=jnp.float32)
    acc += jnp.dot(dst_ref[...], wd_ref[...], preferred_element_type=jnp.float32)
    acc += jnp.dot(dist_ref[...], we_ref[...], preferred_element_type=jnp.float32)
    o_ref[...] = jnp.maximum(acc + b_ref[...], 0.0)


def kernel(node_edge_feat, dist_feat, srcs, dsts, W, b):
    pad = _B_PAD - NUM_EDGES
    srcs_p = jnp.concatenate([srcs, jnp.zeros((pad,), jnp.int32)])
    dsts_p = jnp.concatenate([dsts, jnp.zeros((pad,), jnp.int32)])

    src_g, dst_g = _sc_gather(node_edge_feat, srcs_p, dsts_p)

    ws_t = W[:, :HIDDEN].T
    wd_t = W[:, HIDDEN:2 * HIDDEN].T
    we_t = W[:, 2 * HIDDEN:].T
    b2 = b.reshape(1, HIDDEN)

    feat_spec = pl.BlockSpec((_BLK, HIDDEN), lambda i: (i, 0))
    w_spec = pl.BlockSpec((HIDDEN, HIDDEN), lambda i: (0, 0))
    out = pl.pallas_call(
        _mm_body,
        grid=(NUM_EDGES // _BLK,),
        in_specs=[feat_spec, feat_spec, feat_spec, w_spec, w_spec, w_spec,
                  pl.BlockSpec((1, HIDDEN), lambda i: (0, 0))],
        out_specs=feat_spec,
        out_shape=jax.ShapeDtypeStruct((NUM_EDGES, HIDDEN), jnp.float32),
    )(src_g, dst_g, dist_feat, ws_t, wd_t, we_t, b2)
    return out
